# baseline (device time: 385646 ns/iter reference)
import jax
import jax.numpy as jnp
from jax import lax
from jax.experimental import pallas as pl
from jax.experimental.pallas import tpu as pltpu

N_DEV = 8
M_BLK = 512
HOPS = N_DEV - 1
NC = 1024
LANES = 2
CPL = 2
T = CPL * HOPS


def kernel(x, w_mat):
    x_bf = x.astype(jnp.bfloat16)
    w_bf = w_mat.astype(jnp.bfloat16)
    m, k = x_bf.shape
    _, n = w_bf.shape

    def body(x_ref, w_ref, out_ref, *scratch):
        lane_scratch, scratch = scratch[:20], scratch[20:]
        ax_send, ax_recv, ax_send_sems, ax_recv_sems = scratch
        d = lax.axis_index("i")
        left = lax.rem(d - 1 + N_DEV, N_DEV)
        right = lax.rem(d + 1, N_DEV)

        barrier_sem = pltpu.get_barrier_semaphore()
        for nbr in (left, right):
            pl.semaphore_signal(barrier_sem, inc=1, device_id=(nbr,),
                                device_id_type=pl.DeviceIdType.MESH)
        pl.semaphore_wait(barrier_sem, 2)

        lanes = []
        for li, (cw, par) in enumerate([(True, 0), (False, 0), (True, 1),
                                        (False, 1)]):
            sb, rb, ss, rs, cr = lane_scratch[5 * li:5 * li + 5]
            lanes.append(dict(
                cw=cw, par=par, send=sb, recv=rb, send_sem=ss, recv_sems=rs,
                credit=cr, dst=right if cw else left,
                src=left if cw else right,
                base=(0 if cw else n // 2) + par * NC, rdma={}))

        amax_parts = []

        def step(L, cp, s):
            if L["cw"]:
                j = lax.rem(d - 1 - s + 2 * N_DEV, N_DEV)
            else:
                j = lax.rem(d + 1 + s, N_DEV)
            c0 = L["base"] + cp * LANES * NC
            if s < N_DEV - 1:
                row0 = pl.multiple_of(j * M_BLK, M_BLK)
                partial = jnp.dot(x_ref[pl.ds(row0, M_BLK), :],
                                  w_ref[:, c0:c0 + NC],
                                  preferred_element_type=jnp.float32)
            else:
                partial = out_ref[:, c0:c0 + NC]
            if s == 0:
                acc = partial
            else:
                t_in = HOPS * cp + s - 1
                L["rdma"][t_in].wait_recv()
                acc = partial + L["recv"][t_in % 2, :, :].astype(jnp.float32)
                if t_in <= T - 3:
                    pl.semaphore_signal(L["credit"], inc=1,
                                        device_id=(L["src"],),
                                        device_id_type=pl.DeviceIdType.MESH)
            if s < N_DEV - 1:
                t = HOPS * cp + s
                if t >= 2:
                    pl.semaphore_wait(L["credit"], 1)
                if t >= 1:
                    L["rdma"][t - 1].wait_send()
                L["send"][:, :] = acc.astype(jnp.bfloat16)
                r = pltpu.make_async_remote_copy(
                    src_ref=L["send"],
                    dst_ref=L["recv"].at[t % 2],
                    send_sem=L["send_sem"],
                    recv_sem=L["recv_sems"].at[t % 2],
                    device_id=(L["dst"],),
                    device_id_type=pl.DeviceIdType.MESH,
                )
                r.start()
                L["rdma"][t] = r
                if s == N_DEV - 2:
                    row_d = pl.multiple_of(d * M_BLK, M_BLK)
                    out_ref[:, c0:c0 + NC] = jnp.dot(
                        x_ref[pl.ds(row_d, M_BLK), :], w_ref[:, c0:c0 + NC],
                        preferred_element_type=jnp.float32)
            else:
                relu_acc = jnp.maximum(acc, 0.0)
                out_ref[:, c0:c0 + NC] = relu_acc
                amax_parts.append(jnp.max(relu_acc))

        SHIFT = 4
        NP = N_DEV * CPL
        for p in range(NP + SHIFT):
            for li, L in enumerate(lanes):
                q = p if li < 2 else p - SHIFT
                if 0 <= q < NP:
                    step(L, q // N_DEV, q % N_DEV)

        for L in lanes:
            L["rdma"][T - 1].wait_send()

        local_amax = amax_parts[0]
        for p in amax_parts[1:]:
            local_amax = jnp.maximum(local_amax, p)
        ax_send[:, :] = jnp.full((1, 128), local_amax, jnp.float32)
        rdmas = []
        for kk in range(1, N_DEV):
            tgt = lax.rem(d + kk, N_DEV)
            r = pltpu.make_async_remote_copy(
                src_ref=ax_send,
                dst_ref=ax_recv.at[kk - 1],
                send_sem=ax_send_sems.at[kk - 1],
                recv_sem=ax_recv_sems.at[kk - 1],
                device_id=(tgt,),
                device_id_type=pl.DeviceIdType.MESH,
            )
            r.start()
            rdmas.append(r)
        for r in rdmas:
            r.wait_send()
        for r in rdmas:
            r.wait_recv()
        global_amax = jnp.maximum(local_amax, jnp.max(ax_recv[:, :, :]))

        scale = global_amax / 448.0
        scale_inv = 448.0 / global_amax
        for ci in range(n // NC):
            c0 = ci * NC
            q = (out_ref[:, c0:c0 + NC] * scale_inv).astype(jnp.float8_e4m3fn)
            out_ref[:, c0:c0 + NC] = q.astype(jnp.float32) * scale

    lane_scratch = []
    for _ in range(4):
        lane_scratch += [
            pltpu.VMEM((M_BLK, NC), jnp.bfloat16),
            pltpu.VMEM((2, M_BLK, NC), jnp.bfloat16),
            pltpu.SemaphoreType.DMA,
            pltpu.SemaphoreType.DMA((2,)),
            pltpu.SemaphoreType.REGULAR,
        ]
    return pl.pallas_call(
        body,
        out_shape=jax.ShapeDtypeStruct((M_BLK, n), jnp.float32),
        in_specs=[pl.BlockSpec(memory_space=pltpu.VMEM),
                  pl.BlockSpec(memory_space=pltpu.VMEM)],
        out_specs=pl.BlockSpec(memory_space=pltpu.VMEM),
        scratch_shapes=lane_scratch + [
            pltpu.VMEM((1, 128), jnp.float32),
            pltpu.VMEM((N_DEV - 1, 1, 128), jnp.float32),
            pltpu.SemaphoreType.DMA((N_DEV - 1,)),
            pltpu.SemaphoreType.DMA((N_DEV - 1,)),
        ],
        compiler_params=pltpu.CompilerParams(collective_id=0),
    )(x_bf, w_bf)


# device time: 362346 ns/iter; 1.0643x vs baseline; 1.0643x over previous
import jax
import jax.numpy as jnp
from jax import lax
from jax.experimental import pallas as pl
from jax.experimental.pallas import tpu as pltpu

N_DEV = 8
M_BLK = 512
HOPS = N_DEV - 1
NC = 1024
LANES = 2
CPL = 2
T = CPL * HOPS


def kernel(x, w_mat):
    x_bf = x.astype(jnp.bfloat16)
    w_bf = w_mat.astype(jnp.bfloat16)
    m, k = x_bf.shape
    _, n = w_bf.shape

    def body(x_ref, w_ref, out_ref, *scratch):
        lane_scratch, scratch = scratch[:20], scratch[20:]
        ax_send, ax_recv, ax_send_sems, ax_recv_sems = scratch
        d = lax.axis_index("i")
        left = lax.rem(d - 1 + N_DEV, N_DEV)
        right = lax.rem(d + 1, N_DEV)

        barrier_sem = pltpu.get_barrier_semaphore()
        for nbr in (left, right):
            pl.semaphore_signal(barrier_sem, inc=1, device_id=(nbr,),
                                device_id_type=pl.DeviceIdType.MESH)
        pl.semaphore_wait(barrier_sem, 2)

        lanes = []
        for li, (cw, par) in enumerate([(True, 0), (False, 0), (True, 1),
                                        (False, 1)]):
            sb, rb, ss, rs, cr = lane_scratch[5 * li:5 * li + 5]
            lanes.append(dict(
                cw=cw, par=par, send=sb, recv=rb, send_sem=ss, recv_sems=rs,
                credit=cr, dst=right if cw else left,
                src=left if cw else right,
                base=(0 if cw else n // 2) + par * NC, rdma={}))

        amax_parts = []

        def step(L, cp, s):
            if L["cw"]:
                j = lax.rem(d - 1 - s + 2 * N_DEV, N_DEV)
            else:
                j = lax.rem(d + 1 + s, N_DEV)
            c0 = L["base"] + cp * LANES * NC
            if s < N_DEV - 1:
                row0 = pl.multiple_of(j * M_BLK, M_BLK)
                partial = jnp.dot(x_ref[pl.ds(row0, M_BLK), :],
                                  w_ref[:, c0:c0 + NC],
                                  preferred_element_type=jnp.float32)
            else:
                partial = out_ref[:, c0:c0 + NC]
            if s == 0:
                acc = partial
            else:
                t_in = HOPS * cp + s - 1
                L["rdma"][t_in].wait_recv()
                acc = partial + L["recv"][t_in % 2, :, :].astype(jnp.float32)
                if t_in <= T - 3:
                    pl.semaphore_signal(L["credit"], inc=1,
                                        device_id=(L["src"],),
                                        device_id_type=pl.DeviceIdType.MESH)
            if s < N_DEV - 1:
                t = HOPS * cp + s
                if t >= 2:
                    pl.semaphore_wait(L["credit"], 1)
                if t >= 1:
                    L["rdma"][t - 1].wait_send()
                L["send"][:, :] = acc.astype(jnp.bfloat16)
                r = pltpu.make_async_remote_copy(
                    src_ref=L["send"],
                    dst_ref=L["recv"].at[t % 2],
                    send_sem=L["send_sem"],
                    recv_sem=L["recv_sems"].at[t % 2],
                    device_id=(L["dst"],),
                    device_id_type=pl.DeviceIdType.MESH,
                )
                r.start()
                L["rdma"][t] = r
                if s == N_DEV - 2:
                    row_d = pl.multiple_of(d * M_BLK, M_BLK)
                    out_ref[:, c0:c0 + NC] = jnp.dot(
                        x_ref[pl.ds(row_d, M_BLK), :], w_ref[:, c0:c0 + NC],
                        preferred_element_type=jnp.float32)
            else:
                relu_acc = jnp.maximum(acc, 0.0)
                out_ref[:, c0:c0 + NC] = relu_acc
                amax_parts.append(jnp.max(relu_acc))

        for cp in range(CPL):
            for s in range(N_DEV):
                for L in lanes:
                    step(L, cp, s)

        for L in lanes:
            L["rdma"][T - 1].wait_send()

        local_amax = amax_parts[0]
        for p in amax_parts[1:]:
            local_amax = jnp.maximum(local_amax, p)
        ax_send[:, :] = jnp.full((1, 128), local_amax, jnp.float32)
        rdmas = []
        for kk in range(1, N_DEV):
            tgt = lax.rem(d + kk, N_DEV)
            r = pltpu.make_async_remote_copy(
                src_ref=ax_send,
                dst_ref=ax_recv.at[kk - 1],
                send_sem=ax_send_sems.at[kk - 1],
                recv_sem=ax_recv_sems.at[kk - 1],
                device_id=(tgt,),
                device_id_type=pl.DeviceIdType.MESH,
            )
            r.start()
            rdmas.append(r)
        for r in rdmas:
            r.wait_send()
        for r in rdmas:
            r.wait_recv()
        global_amax = jnp.maximum(local_amax, jnp.max(ax_recv[:, :, :]))

        scale = global_amax / 448.0
        scale_inv = 448.0 / global_amax
        for ci in range(n // NC):
            c0 = ci * NC
            q = (out_ref[:, c0:c0 + NC] * scale_inv).astype(jnp.float8_e4m3fn)
            out_ref[:, c0:c0 + NC] = q.astype(jnp.float32) * scale

    lane_scratch = []
    for _ in range(4):
        lane_scratch += [
            pltpu.VMEM((M_BLK, NC), jnp.bfloat16),
            pltpu.VMEM((2, M_BLK, NC), jnp.bfloat16),
            pltpu.SemaphoreType.DMA,
            pltpu.SemaphoreType.DMA((2,)),
            pltpu.SemaphoreType.REGULAR,
        ]
    return pl.pallas_call(
        body,
        out_shape=jax.ShapeDtypeStruct((M_BLK, n), jnp.float32),
        in_specs=[pl.BlockSpec(memory_space=pltpu.VMEM),
                  pl.BlockSpec(memory_space=pltpu.VMEM)],
        out_specs=pl.BlockSpec(memory_space=pltpu.VMEM),
        scratch_shapes=lane_scratch + [
            pltpu.VMEM((1, 128), jnp.float32),
            pltpu.VMEM((N_DEV - 1, 1, 128), jnp.float32),
            pltpu.SemaphoreType.DMA((N_DEV - 1,)),
            pltpu.SemaphoreType.DMA((N_DEV - 1,)),
        ],
        compiler_params=pltpu.CompilerParams(collective_id=0),
    )(x_bf, w_bf)
